# Initial kernel scaffold; baseline (speedup 1.0000x reference)
import jax
import jax.numpy as jnp
from jax.experimental import pallas as pl


def kernel(x, edge_index, Ws1, Wn1, b1, Ws2, Wn2, b2):
    raise NotImplementedError("write your pallas kernel here")


# speedup vs baseline: 5.0095x; 5.0095x over previous
import functools

import jax
import jax.numpy as jnp
from jax import lax
from jax.experimental import pallas as pl
from jax.experimental.pallas import tpu as pltpu
from jax.experimental.pallas import tpu_sc as plsc

_N = 10000
_E = 320000
_D = 128
_NP = 10240
_NC = 2
_NS = 16
_NW = _NC * _NS
_EPW = _E // _NW
_K = 80
_NCHUNK = _EPW // _K
_RPT = _NP // _NS


def _sc_agg_body(h_hbm, src_hbm, dst_hbm, agg_hbm, deg_hbm,
                 srcb, dstb, rows, ones, zrows, sh_agg, sh_deg, gsem):
    c = lax.axis_index("c")
    s = lax.axis_index("s")
    wid = s * _NC + c

    zro = jnp.zeros((16,), jnp.float32)
    one = jnp.ones((16,), jnp.float32)
    for j in range(_K // 16):
        ones[pl.ds(j * 16, 16)] = one

    def _zrow(i, carry):
        for j in range(_D // 16):
            zrows[i, pl.ds(j * 16, 16)] = zro
        return carry
    lax.fori_loop(0, _K, _zrow, 0)

    row0 = s * _RPT
    for r in range(_RPT // _K):
        pltpu.sync_copy(zrows, sh_agg.at[pl.ds(row0 + r * _K, _K)])
    for j in range(_RPT // _D):
        pltpu.sync_copy(zrows.at[0], sh_deg.at[pl.ds(row0 + j * _D, _D)])
    plsc.subcore_barrier()

    ebase = wid * _EPW

    def _chunk(g, carry):
        base = ebase + g * _K
        pltpu.sync_copy(src_hbm.at[pl.ds(base, _K)], srcb)
        pltpu.sync_copy(dst_hbm.at[pl.ds(base, _K)], dstb)
        pltpu.async_copy(h_hbm.at[srcb], rows, gsem).wait()
        pltpu.sync_copy(rows, sh_agg.at[dstb], add=True)
        pltpu.sync_copy(ones, sh_deg.at[dstb], add=True)
        return carry
    lax.fori_loop(0, _NCHUNK, _chunk, 0)
    plsc.subcore_barrier()

    pltpu.sync_copy(sh_agg.at[pl.ds(row0, _RPT)], agg_hbm.at[c, pl.ds(row0, _RPT)])
    pltpu.sync_copy(sh_deg.at[pl.ds(row0, _RPT)], deg_hbm.at[c, pl.ds(row0, _RPT)])


_sc_agg = functools.partial(
    pl.kernel,
    out_type=(jax.ShapeDtypeStruct((_NC, _NP, _D), jnp.float32),
              jax.ShapeDtypeStruct((_NC, _NP), jnp.float32)),
    mesh=plsc.VectorSubcoreMesh(core_axis_name="c", subcore_axis_name="s"),
    scratch_types=[
        pltpu.VMEM((_K,), jnp.int32),
        pltpu.VMEM((_K,), jnp.int32),
        pltpu.VMEM((_K, _D), jnp.float32),
        pltpu.VMEM((_K,), jnp.float32),
        pltpu.VMEM((_K, _D), jnp.float32),
        pltpu.VMEM_SHARED((_NP, _D), jnp.float32),
        pltpu.VMEM_SHARED((_NP,), jnp.float32),
        pltpu.SemaphoreType.DMA,
    ],
)(_sc_agg_body)


def _tc_layer(h, agg, dA, dB, Ws, Wn, b, relu):
    BN = 1000

    def body(h_ref, a0_ref, a1_ref, dA_ref, dB_ref, Ws_ref, Wn_ref, b_ref, o_ref):
        deg = jnp.maximum(dA_ref[...] + dB_ref[...], 1.0)
        hn = (a0_ref[0] + a1_ref[0]) / deg
        out = (jnp.dot(h_ref[...], Ws_ref[...], preferred_element_type=jnp.float32)
               + jnp.dot(hn, Wn_ref[...], preferred_element_type=jnp.float32)
               + b_ref[...])
        if relu:
            out = jnp.maximum(out, 0.0)
        o_ref[...] = out

    return pl.pallas_call(
        body,
        grid=(_N // BN,),
        in_specs=[
            pl.BlockSpec((BN, _D), lambda i: (i, 0)),
            pl.BlockSpec((1, BN, _D), lambda i: (0, i, 0)),
            pl.BlockSpec((1, BN, _D), lambda i: (1, i, 0)),
            pl.BlockSpec((BN, 1), lambda i: (i, 0)),
            pl.BlockSpec((BN, 1), lambda i: (i, 0)),
            pl.BlockSpec((_D, _D), lambda i: (0, 0)),
            pl.BlockSpec((_D, _D), lambda i: (0, 0)),
            pl.BlockSpec((1, _D), lambda i: (0, 0)),
        ],
        out_specs=pl.BlockSpec((BN, _D), lambda i: (i, 0)),
        out_shape=jax.ShapeDtypeStruct((_N, _D), jnp.float32),
    )(h, agg, agg, dA, dB, Ws, Wn, b)


def kernel(x, edge_index, Ws1, Wn1, b1, Ws2, Wn2, b2):
    src = edge_index[0]
    dst = edge_index[1]
    agg1, deg = _sc_agg(x, src, dst)
    dA = deg[0, :_N, None]
    dB = deg[1, :_N, None]
    h1 = _tc_layer(x, agg1, dA, dB, Ws1, Wn1, b1.reshape(1, _D), relu=True)
    agg2, _ = _sc_agg(h1, src, dst)
    return _tc_layer(h1, agg2, dA, dB, Ws2, Wn2, b2.reshape(1, _D), relu=False)


# speedup vs baseline: 9.3297x; 1.8624x over previous
import functools

import jax
import jax.numpy as jnp
from jax import lax
from jax.experimental import pallas as pl
from jax.experimental.pallas import tpu as pltpu
from jax.experimental.pallas import tpu_sc as plsc

_N = 10000
_E = 320000
_D = 128
_NP = 10240
_NC = 2
_NS = 16
_NW = _NC * _NS
_EPW = _E // _NW
_K = 80
_NCHUNK = _EPW // _K
_RPT = _NP // _NS
_ZR = 16


def _sc_agg_body(want_deg, h_hbm, src_hbm, dst_hbm, agg_hbm, deg_hbm,
                 srcb, dstb, rows0, rows1, ones, zrows, sh_agg, sh_deg,
                 sem0, sem1):
    c = lax.axis_index("c")
    s = lax.axis_index("s")
    wid = s * _NC + c

    zro = jnp.zeros((16,), jnp.float32)
    one = jnp.ones((16,), jnp.float32)
    for j in range(_K // 16):
        ones[pl.ds(j * 16, 16)] = one

    def _zrow(i, carry):
        for j in range(_D // 16):
            zrows[i, pl.ds(j * 16, 16)] = zro
        return carry
    lax.fori_loop(0, _ZR, _zrow, 0)

    pltpu.sync_copy(src_hbm.at[pl.ds(wid * _EPW, _EPW)], srcb)
    pltpu.sync_copy(dst_hbm.at[pl.ds(wid * _EPW, _EPW)], dstb)

    row0 = s * _RPT
    for r in range(_RPT // _ZR):
        pltpu.sync_copy(zrows, sh_agg.at[pl.ds(row0 + r * _ZR, _ZR)])
    if want_deg:
        for j in range(_RPT // _D):
            pltpu.sync_copy(zrows.at[0], sh_deg.at[pl.ds(row0 + j * _D, _D)])
    plsc.subcore_barrier()

    rows = (rows0, rows1)
    sems = (sem0, sem1)

    def _sidx(g):
        return srcb.at[pl.ds(g * _K, _K)]

    def _didx(g):
        return dstb.at[pl.ds(g * _K, _K)]

    def _scatter(g, b):
        pltpu.sync_copy(rows[b], sh_agg.at[_didx(g)], add=True)
        if want_deg:
            pltpu.sync_copy(ones, sh_deg.at[_didx(g)], add=True)

    pltpu.async_copy(h_hbm.at[_sidx(0)], rows[0], sems[0])

    def _run():
        def body(i, carry):
            g = i * 2
            pltpu.make_async_copy(h_hbm.at[_sidx(g)], rows[0], sems[0]).wait()
            pltpu.async_copy(h_hbm.at[_sidx(g + 1)], rows[1], sems[1])
            _scatter(g, 0)
            pltpu.make_async_copy(h_hbm.at[_sidx(g + 1)], rows[1], sems[1]).wait()
            @pl.when(i < _NCHUNK // 2 - 1)
            def _():
                pltpu.async_copy(h_hbm.at[_sidx(g + 2)], rows[0], sems[0])
            _scatter(g + 1, 1)
            return carry
        lax.fori_loop(0, _NCHUNK // 2, body, 0)
        last = _NCHUNK - 1
        pltpu.async_copy(h_hbm.at[_sidx(last)], rows[0], sems[0]).wait()
        _scatter(last, 0)

    _run()
    plsc.subcore_barrier()

    pltpu.sync_copy(sh_agg.at[pl.ds(row0, _RPT)], agg_hbm.at[c, pl.ds(row0, _RPT)])
    if want_deg:
        pltpu.sync_copy(sh_deg.at[pl.ds(row0, _RPT)], deg_hbm.at[c, pl.ds(row0, _RPT)])


def _make_sc_agg(want_deg):
    body = functools.partial(_sc_agg_body, want_deg)
    return functools.partial(
        pl.kernel,
        out_type=(jax.ShapeDtypeStruct((_NC, _NP, _D), jnp.float32),
                  jax.ShapeDtypeStruct((_NC, _NP), jnp.float32)),
        mesh=plsc.VectorSubcoreMesh(core_axis_name="c", subcore_axis_name="s"),
        scratch_types=[
            pltpu.VMEM((_EPW,), jnp.int32),
            pltpu.VMEM((_EPW,), jnp.int32),
            pltpu.VMEM((_K, _D), jnp.float32),
            pltpu.VMEM((_K, _D), jnp.float32),
            pltpu.VMEM((_K,), jnp.float32),
            pltpu.VMEM((_ZR, _D), jnp.float32),
            pltpu.VMEM_SHARED((_NP, _D), jnp.float32),
            pltpu.VMEM_SHARED((_NP,), jnp.float32),
            pltpu.SemaphoreType.DMA,
            pltpu.SemaphoreType.DMA,
        ],
    )(body)


_sc_agg = _make_sc_agg(True)
_sc_agg_nodeg = _make_sc_agg(False)


def _tc_layer(h, agg, dA, dB, Ws, Wn, b, relu):
    BN = 1000

    def body(h_ref, a0_ref, a1_ref, dA_ref, dB_ref, Ws_ref, Wn_ref, b_ref, o_ref):
        deg = jnp.maximum(dA_ref[...] + dB_ref[...], 1.0)
        hn = (a0_ref[0] + a1_ref[0]) / deg
        out = (jnp.dot(h_ref[...], Ws_ref[...], preferred_element_type=jnp.float32)
               + jnp.dot(hn, Wn_ref[...], preferred_element_type=jnp.float32)
               + b_ref[...])
        if relu:
            out = jnp.maximum(out, 0.0)
        o_ref[...] = out

    return pl.pallas_call(
        body,
        grid=(_N // BN,),
        in_specs=[
            pl.BlockSpec((BN, _D), lambda i: (i, 0)),
            pl.BlockSpec((1, BN, _D), lambda i: (0, i, 0)),
            pl.BlockSpec((1, BN, _D), lambda i: (1, i, 0)),
            pl.BlockSpec((BN, 1), lambda i: (i, 0)),
            pl.BlockSpec((BN, 1), lambda i: (i, 0)),
            pl.BlockSpec((_D, _D), lambda i: (0, 0)),
            pl.BlockSpec((_D, _D), lambda i: (0, 0)),
            pl.BlockSpec((1, _D), lambda i: (0, 0)),
        ],
        out_specs=pl.BlockSpec((BN, _D), lambda i: (i, 0)),
        out_shape=jax.ShapeDtypeStruct((_N, _D), jnp.float32),
    )(h, agg, agg, dA, dB, Ws, Wn, b)


def kernel(x, edge_index, Ws1, Wn1, b1, Ws2, Wn2, b2):
    src = edge_index[0]
    dst = edge_index[1]
    agg1, deg = _sc_agg(x, src, dst)
    dA = deg[0, :_N, None]
    dB = deg[1, :_N, None]
    h1 = _tc_layer(x, agg1, dA, dB, Ws1, Wn1, b1.reshape(1, _D), relu=True)
    agg2, _ = _sc_agg_nodeg(h1, src, dst)
    return _tc_layer(h1, agg2, dA, dB, Ws2, Wn2, b2.reshape(1, _D), relu=False)


# speedup vs baseline: 11.3599x; 1.2176x over previous
import functools

import jax
import jax.numpy as jnp
from jax import lax
from jax.experimental import pallas as pl
from jax.experimental.pallas import tpu as pltpu
from jax.experimental.pallas import tpu_sc as plsc

_N = 10000
_E = 320000
_D = 128
_NP = 10240
_NC = 2
_NS = 16
_NW = _NC * _NS
_EPW = _E // _NW
_K = 80
_NCHUNK = _EPW // _K
_RPT = _NP // _NS
_ZR = 16


def _sc_agg_body(want_deg, h_hbm, src_hbm, dst_hbm, agg_hbm, deg_hbm,
                 srcb, dstb, rows0, rows1, ones, zrows, sh_agg, sh_deg,
                 sem0, sem1, sem2, sem3):
    c = lax.axis_index("c")
    s = lax.axis_index("s")
    wid = s * _NC + c

    zro = jnp.zeros((16,), jnp.float32)
    one = jnp.ones((16,), jnp.float32)
    for j in range(_K // 16):
        ones[pl.ds(j * 16, 16)] = one

    def _zrow(i, carry):
        for j in range(_D // 16):
            zrows[i, pl.ds(j * 16, 16)] = zro
        return carry
    lax.fori_loop(0, _ZR, _zrow, 0)

    pltpu.sync_copy(src_hbm.at[pl.ds(wid * _EPW, _EPW)], srcb)
    pltpu.sync_copy(dst_hbm.at[pl.ds(wid * _EPW, _EPW)], dstb)

    row0 = s * _RPT
    for r in range(_RPT // _ZR):
        pltpu.sync_copy(zrows, sh_agg.at[pl.ds(row0 + r * _ZR, _ZR)])
    if want_deg:
        for j in range(_RPT // _D):
            pltpu.sync_copy(zrows.at[0], sh_deg.at[pl.ds(row0 + j * _D, _D)])
    plsc.subcore_barrier()

    rows = (rows0, rows1)
    gsems = (sem0, sem1)
    ssems = (sem2, sem3)

    def _sidx(g):
        return srcb.at[pl.ds(g * _K, _K)]

    def _didx(g):
        return dstb.at[pl.ds(g * _K, _K)]

    def _gather(g, b):
        pltpu.async_copy(h_hbm.at[_sidx(g)], rows[b], gsems[b])

    def _wait_gather(g, b):
        pltpu.make_async_copy(h_hbm.at[_sidx(g)], rows[b], gsems[b]).wait()

    def _scatter(g, b):
        pltpu.async_copy(rows[b], sh_agg.at[_didx(g)], ssems[b], add=True)
        if want_deg:
            pltpu.sync_copy(ones, sh_deg.at[_didx(g)], add=True)

    def _wait_scatter(g, b):
        pltpu.make_async_copy(rows[b], sh_agg.at[_didx(g)], ssems[b]).wait()

    _gather(0, 0)

    def _run():
        def body(i, carry):
            g = i * 2
            @pl.when(i > 0)
            def _():
                _wait_scatter(g - 1, 1)
            _gather(g + 1, 1)
            _wait_gather(g, 0)
            _scatter(g, 0)
            _wait_gather(g + 1, 1)
            _wait_scatter(g, 0)
            @pl.when(i < _NCHUNK // 2 - 1)
            def _():
                _gather(g + 2, 0)
            _scatter(g + 1, 1)
            return carry
        lax.fori_loop(0, _NCHUNK // 2, body, 0)
        last = _NCHUNK - 1
        _wait_scatter(last - 1, 1)
        _gather(last, 0)
        _wait_gather(last, 0)
        _scatter(last, 0)
        _wait_scatter(last, 0)

    _run()
    plsc.subcore_barrier()

    pltpu.sync_copy(sh_agg.at[pl.ds(row0, _RPT)], agg_hbm.at[c, pl.ds(row0, _RPT)])
    if want_deg:
        pltpu.sync_copy(sh_deg.at[pl.ds(row0, _RPT)], deg_hbm.at[c, pl.ds(row0, _RPT)])


def _make_sc_agg(want_deg):
    body = functools.partial(_sc_agg_body, want_deg)
    return functools.partial(
        pl.kernel,
        out_type=(jax.ShapeDtypeStruct((_NC, _NP, _D), jnp.float32),
                  jax.ShapeDtypeStruct((_NC, _NP), jnp.float32)),
        mesh=plsc.VectorSubcoreMesh(core_axis_name="c", subcore_axis_name="s"),
        scratch_types=[
            pltpu.VMEM((_EPW,), jnp.int32),
            pltpu.VMEM((_EPW,), jnp.int32),
            pltpu.VMEM((_K, _D), jnp.float32),
            pltpu.VMEM((_K, _D), jnp.float32),
            pltpu.VMEM((_K,), jnp.float32),
            pltpu.VMEM((_ZR, _D), jnp.float32),
            pltpu.VMEM_SHARED((_NP, _D), jnp.float32),
            pltpu.VMEM_SHARED((_NP,), jnp.float32),
            pltpu.SemaphoreType.DMA,
            pltpu.SemaphoreType.DMA,
            pltpu.SemaphoreType.DMA,
            pltpu.SemaphoreType.DMA,
        ],
    )(body)


_sc_agg = _make_sc_agg(True)
_sc_agg_nodeg = _make_sc_agg(False)


def _tc_layer(h, agg, dA, dB, Ws, Wn, b, relu):
    BN = 1000

    def body(h_ref, a0_ref, a1_ref, dA_ref, dB_ref, Ws_ref, Wn_ref, b_ref, o_ref):
        deg = jnp.maximum(dA_ref[...] + dB_ref[...], 1.0)
        hn = (a0_ref[0] + a1_ref[0]) / deg
        out = (jnp.dot(h_ref[...], Ws_ref[...], preferred_element_type=jnp.float32)
               + jnp.dot(hn, Wn_ref[...], preferred_element_type=jnp.float32)
               + b_ref[...])
        if relu:
            out = jnp.maximum(out, 0.0)
        o_ref[...] = out

    return pl.pallas_call(
        body,
        grid=(_N // BN,),
        in_specs=[
            pl.BlockSpec((BN, _D), lambda i: (i, 0)),
            pl.BlockSpec((1, BN, _D), lambda i: (0, i, 0)),
            pl.BlockSpec((1, BN, _D), lambda i: (1, i, 0)),
            pl.BlockSpec((BN, 1), lambda i: (i, 0)),
            pl.BlockSpec((BN, 1), lambda i: (i, 0)),
            pl.BlockSpec((_D, _D), lambda i: (0, 0)),
            pl.BlockSpec((_D, _D), lambda i: (0, 0)),
            pl.BlockSpec((1, _D), lambda i: (0, 0)),
        ],
        out_specs=pl.BlockSpec((BN, _D), lambda i: (i, 0)),
        out_shape=jax.ShapeDtypeStruct((_N, _D), jnp.float32),
    )(h, agg, agg, dA, dB, Ws, Wn, b)


def kernel(x, edge_index, Ws1, Wn1, b1, Ws2, Wn2, b2):
    src = edge_index[0]
    dst = edge_index[1]
    agg1, deg = _sc_agg(x, src, dst)
    dA = deg[0, :_N, None]
    dB = deg[1, :_N, None]
    h1 = _tc_layer(x, agg1, dA, dB, Ws1, Wn1, b1.reshape(1, _D), relu=True)
    agg2, _ = _sc_agg_nodeg(h1, src, dst)
    return _tc_layer(h1, agg2, dA, dB, Ws2, Wn2, b2.reshape(1, _D), relu=False)


# speedup vs baseline: 12.4698x; 1.0977x over previous
import functools

import jax
import jax.numpy as jnp
from jax import lax
from jax.experimental import pallas as pl
from jax.experimental.pallas import tpu as pltpu
from jax.experimental.pallas import tpu_sc as plsc

_N = 10000
_E = 320000
_D = 128
_NP = 10240
_NC = 2
_NS = 16
_NW = _NC * _NS
_EPW = _E // _NW
_K = 40
_NCHUNK = _EPW // _K
_RPT = _NP // _NS
_ZR = 16


def _sc_agg_body(want_deg, h_hbm, src_hbm, dst_hbm, agg_hbm, deg_hbm,
                 srcb, dstb, rows0, rows1, rows2, rows3, rows4, ones, zrows,
                 sh_agg, sh_deg,
                 gsem0, gsem1, gsem2, gsem3, gsem4,
                 ssem0, ssem1, ssem2, ssem3, ssem4):
    c = lax.axis_index("c")
    s = lax.axis_index("s")
    wid = s * _NC + c

    zro = jnp.zeros((16,), jnp.float32)
    one = jnp.ones((16,), jnp.float32)
    for j in range((_K + 15) // 16):
        ones[pl.ds(min(j * 16, _K - 16), 16)] = one

    def _zrow(i, carry):
        for j in range(_D // 16):
            zrows[i, pl.ds(j * 16, 16)] = zro
        return carry
    lax.fori_loop(0, _ZR, _zrow, 0)

    pltpu.sync_copy(src_hbm.at[pl.ds(wid * _EPW, _EPW)], srcb)
    pltpu.sync_copy(dst_hbm.at[pl.ds(wid * _EPW, _EPW)], dstb)

    row0 = s * _RPT
    for r in range(_RPT // _ZR):
        pltpu.sync_copy(zrows, sh_agg.at[pl.ds(row0 + r * _ZR, _ZR)])
    if want_deg:
        for j in range(_RPT // _D):
            pltpu.sync_copy(zrows.at[0], sh_deg.at[pl.ds(row0 + j * _D, _D)])
    plsc.subcore_barrier()

    rows = (rows0, rows1, rows2, rows3, rows4)
    gsems = (gsem0, gsem1, gsem2, gsem3, gsem4)
    ssems = (ssem0, ssem1, ssem2, ssem3, ssem4)

    def _sidx(g):
        return srcb.at[pl.ds(g * _K, _K)]

    def _didx(g):
        return dstb.at[pl.ds(g * _K, _K)]

    def _gather(g, b):
        pltpu.async_copy(h_hbm.at[_sidx(g)], rows[b], gsems[b])

    def _wait_gather(g, b):
        pltpu.make_async_copy(h_hbm.at[_sidx(g)], rows[b], gsems[b]).wait()

    def _scatter(g, b):
        pltpu.async_copy(rows[b], sh_agg.at[_didx(g)], ssems[b], add=True)
        if want_deg:
            pltpu.sync_copy(ones, sh_deg.at[_didx(g)], add=True)

    def _wait_scatter(g, b):
        pltpu.make_async_copy(rows[b], sh_agg.at[_didx(g)], ssems[b]).wait()

    _gather(0, 0)
    _gather(1, 1)

    def _run():
        def body(j, carry):
            for b in range(5):
                g = j * 5 + b
                b2 = (b + 2) % 5
                @pl.when(g >= 3)
                def _():
                    _wait_scatter(g - 3, b2)
                @pl.when(g + 2 <= _NCHUNK - 1)
                def _():
                    _gather(g + 2, b2)
                _wait_gather(g, b)
                _scatter(g, b)
            return carry
        lax.fori_loop(0, _NCHUNK // 5, body, 0)
        for t in range(3):
            g = _NCHUNK - 3 + t
            _wait_scatter(g, g % 5)

    _run()
    plsc.subcore_barrier()

    pltpu.sync_copy(sh_agg.at[pl.ds(row0, _RPT)], agg_hbm.at[c, pl.ds(row0, _RPT)])
    if want_deg:
        pltpu.sync_copy(sh_deg.at[pl.ds(row0, _RPT)], deg_hbm.at[c, pl.ds(row0, _RPT)])


def _make_sc_agg(want_deg):
    body = functools.partial(_sc_agg_body, want_deg)
    return functools.partial(
        pl.kernel,
        out_type=(jax.ShapeDtypeStruct((_NC, _NP, _D), jnp.float32),
                  jax.ShapeDtypeStruct((_NC, _NP), jnp.float32)),
        mesh=plsc.VectorSubcoreMesh(core_axis_name="c", subcore_axis_name="s"),
        scratch_types=[
            pltpu.VMEM((_EPW,), jnp.int32),
            pltpu.VMEM((_EPW,), jnp.int32),
            pltpu.VMEM((_K, _D), jnp.float32),
            pltpu.VMEM((_K, _D), jnp.float32),
            pltpu.VMEM((_K, _D), jnp.float32),
            pltpu.VMEM((_K, _D), jnp.float32),
            pltpu.VMEM((_K, _D), jnp.float32),
            pltpu.VMEM((_K,), jnp.float32),
            pltpu.VMEM((_ZR, _D), jnp.float32),
            pltpu.VMEM_SHARED((_NP, _D), jnp.float32),
            pltpu.VMEM_SHARED((_NP,), jnp.float32),
        ] + [pltpu.SemaphoreType.DMA] * 10,
    )(body)


_sc_agg = _make_sc_agg(True)
_sc_agg_nodeg = _make_sc_agg(False)


def _tc_layer(h, agg, dA, dB, Ws, Wn, b, relu):
    BN = 1000

    def body(h_ref, a0_ref, a1_ref, dA_ref, dB_ref, Ws_ref, Wn_ref, b_ref, o_ref):
        deg = jnp.maximum(dA_ref[...] + dB_ref[...], 1.0)
        hn = (a0_ref[0] + a1_ref[0]) / deg
        out = (jnp.dot(h_ref[...], Ws_ref[...], preferred_element_type=jnp.float32)
               + jnp.dot(hn, Wn_ref[...], preferred_element_type=jnp.float32)
               + b_ref[...])
        if relu:
            out = jnp.maximum(out, 0.0)
        o_ref[...] = out

    return pl.pallas_call(
        body,
        grid=(_N // BN,),
        in_specs=[
            pl.BlockSpec((BN, _D), lambda i: (i, 0)),
            pl.BlockSpec((1, BN, _D), lambda i: (0, i, 0)),
            pl.BlockSpec((1, BN, _D), lambda i: (1, i, 0)),
            pl.BlockSpec((BN, 1), lambda i: (i, 0)),
            pl.BlockSpec((BN, 1), lambda i: (i, 0)),
            pl.BlockSpec((_D, _D), lambda i: (0, 0)),
            pl.BlockSpec((_D, _D), lambda i: (0, 0)),
            pl.BlockSpec((1, _D), lambda i: (0, 0)),
        ],
        out_specs=pl.BlockSpec((BN, _D), lambda i: (i, 0)),
        out_shape=jax.ShapeDtypeStruct((_N, _D), jnp.float32),
    )(h, agg, agg, dA, dB, Ws, Wn, b)


def kernel(x, edge_index, Ws1, Wn1, b1, Ws2, Wn2, b2):
    src = edge_index[0]
    dst = edge_index[1]
    agg1, deg = _sc_agg(x, src, dst)
    dA = deg[0, :_N, None]
    dB = deg[1, :_N, None]
    h1 = _tc_layer(x, agg1, dA, dB, Ws1, Wn1, b1.reshape(1, _D), relu=True)
    agg2, _ = _sc_agg_nodeg(h1, src, dst)
    return _tc_layer(h1, agg2, dA, dB, Ws2, Wn2, b2.reshape(1, _D), relu=False)
